# baseline (speedup 1.0000x reference)
import functools

import jax
import jax.numpy as jnp
from jax.experimental import pallas as pl
from jax.experimental.pallas import tpu as pltpu

K_TOP = 2


def _router_body(x_ref, wrt_ref, br_ref, g_ref):
    logits = jnp.dot(x_ref[...], wrt_ref[...],
                     preferred_element_type=jnp.float32) + br_ref[...]
    w = jax.nn.softmax(logits, axis=-1)
    tn, e = w.shape
    eidx = jax.lax.broadcasted_iota(jnp.int32, (1, e), 1)
    rank = jnp.zeros((tn, e), jnp.int32)
    for ep in range(e):
        wp = w[:, ep:ep + 1]
        beats = (wp > w) | ((wp == w) & (ep < eidx))
        rank = rank + beats.astype(jnp.int32)
    sel = rank < K_TOP
    denom = jnp.sum(jnp.where(sel, w, 0.0), axis=-1, keepdims=True)
    g_ref[...] = jnp.where(sel, w / denom, 0.0)


def _router(xf, Wr, br):
    n, d = xf.shape
    e = Wr.shape[0]
    tn = 1024
    return pl.pallas_call(
        _router_body,
        grid=(n // tn,),
        in_specs=[
            pl.BlockSpec((tn, d), lambda t: (t, 0)),
            pl.BlockSpec((d, e), lambda t: (0, 0)),
            pl.BlockSpec((1, e), lambda t: (0, 0)),
        ],
        out_specs=pl.BlockSpec((tn, e), lambda t: (t, 0)),
        out_shape=jax.ShapeDtypeStruct((n, e), jnp.float32),
    )(xf, Wr.T, br.reshape(1, e))


def _moe_body(g_ref, x_ref, w1_ref, b1_ref, w2_ref, b2_ref, out_ref, *, tm):
    p, mt, t = pl.program_id(0), pl.program_id(1), pl.program_id(2)
    rows = pl.ds(t * tm, tm)
    x_t = x_ref[rows, :]
    a = jnp.dot(x_t, w1_ref[0].T, preferred_element_type=jnp.float32)
    a = a + b1_ref[0]
    h = (a * jax.nn.sigmoid(a)).astype(jnp.bfloat16)
    y = jax.lax.dot_general(h, w2_ref[0], (((1,), (1,)), ((), ())),
                            preferred_element_type=jnp.float32)
    g = g_ref[0, 0, rows]
    contrib = g[:, None] * y

    @pl.when((mt == 0))
    def _():
        contrib2 = contrib + g[:, None] * b2_ref[0]

        @pl.when(p == 0)
        def _():
            out_ref[rows, :] = contrib2

        @pl.when(p > 0)
        def _():
            out_ref[rows, :] += contrib2

    @pl.when(mt > 0)
    def _():
        out_ref[rows, :] += contrib


def _moe_dense(gall, xf, W1all, b1all, W2all, b2all):
    n, d = xf.shape
    p, m, _ = W1all.shape
    tn = 512
    tm = 1024
    mt = m // tn
    kern = functools.partial(_moe_body, tm=tm)
    return pl.pallas_call(
        kern,
        grid=(p, mt, n // tm),
        in_specs=[
            pl.BlockSpec((1, 1, n), lambda p, mt, t: (p, 0, 0)),
            pl.BlockSpec((n, d), lambda p, mt, t: (0, 0)),
            pl.BlockSpec((1, tn, d), lambda p, mt, t: (p, mt, 0)),
            pl.BlockSpec((1, 1, tn), lambda p, mt, t: (p, 0, mt)),
            pl.BlockSpec((1, d, tn), lambda p, mt, t: (p, 0, mt)),
            pl.BlockSpec((1, 1, d), lambda p, mt, t: (p, 0, 0)),
        ],
        out_specs=pl.BlockSpec((n, d), lambda p, mt, t: (0, 0)),
        out_shape=jax.ShapeDtypeStruct((n, d), jnp.float32),
    )(gall[:, None, :], xf, W1all, b1all[:, None, :], W2all,
      b2all[:, None, :])


def kernel(x, Wr, br, W1, b1, W2, b2, Ws1, bs1, Ws2, bs2):
    b, s, d = x.shape
    e, m, _ = W1.shape
    ns, i, _ = Ws1.shape
    n = b * s
    nc = i // m
    xf = x.reshape(n, d)

    gates = _router(xf, Wr, br)

    W1all = jnp.concatenate([W1, Ws1.reshape(ns * nc, m, d)], axis=0)
    b1all = jnp.concatenate([b1, bs1.reshape(ns * nc, m)], axis=0)
    W2all = jnp.concatenate(
        [W2, Ws2.reshape(ns, d, nc, m).transpose(0, 2, 1, 3).reshape(ns * nc, d, m)],
        axis=0)
    bs2_chunks = jnp.concatenate(
        [bs2[:, None, :], jnp.zeros((ns, nc - 1, d), bs2.dtype)], axis=1)
    b2all = jnp.concatenate([b2, bs2_chunks.reshape(ns * nc, d)], axis=0)
    gall = jnp.concatenate(
        [gates.T, jnp.ones((ns * nc, n), jnp.float32)], axis=0)

    out = _moe_dense(gall, xf.astype(jnp.bfloat16),
                     W1all.astype(jnp.bfloat16), b1all,
                     W2all.astype(jnp.bfloat16), b2all)
    return out.reshape(b, s, d)


# speedup vs baseline: 1.8076x; 1.8076x over previous
import functools

import jax
import jax.numpy as jnp
from jax import lax
from jax.experimental import pallas as pl
from jax.experimental.pallas import tpu as pltpu
from jax.experimental.pallas import tpu_sc as plsc

K_TOP = 2
TM = 256


def _router_body(x_ref, wrt_ref, br_ref, gk_ref, cnt_ref, pos_ref):
    logits = jnp.dot(x_ref[...], wrt_ref[...],
                     preferred_element_type=jnp.float32) + br_ref[...]
    w = jax.nn.softmax(logits, axis=-1)
    n, e = w.shape
    eidx = lax.broadcasted_iota(jnp.int32, (1, e), 1)
    rank = jnp.zeros((n, e), jnp.int32)
    for ep in range(e):
        wp = w[:, ep:ep + 1]
        beats = (wp > w) | ((wp == w) & (ep < eidx))
        rank = rank + beats.astype(jnp.int32)
    sel = rank < K_TOP
    denom = jnp.sum(jnp.where(sel, w, 0.0), axis=-1, keepdims=True)
    gdense = jnp.where(sel, w / denom, 0.0)
    for k in range(K_TOP):
        gk_ref[k, :] = jnp.sum(jnp.where(rank == k, gdense, 0.0), axis=-1)

    self = sel.astype(jnp.int32)
    cnt = jnp.sum(self, axis=0, keepdims=True)
    cnt_ref[...] = cnt
    cntf = cnt.astype(jnp.float32)
    offs = [jnp.zeros((1, 1), jnp.float32)]
    for ee in range(e - 1):
        offs.append(offs[-1] + cntf[:, ee:ee + 1])
    off = jnp.concatenate(offs, axis=1)
    run = self
    sft = 1
    while sft < n:
        run = run + jnp.concatenate(
            [jnp.zeros((sft, e), run.dtype), run[:n - sft]], axis=0)
        sft *= 2
    run = (run - self).astype(jnp.float32)
    posmat = off + run
    for k in range(K_TOP):
        pos_ref[k, :] = jnp.sum(jnp.where(rank == k, posmat, 0.0), axis=-1)


def _router(xf, Wr, br):
    n, d = xf.shape
    e = Wr.shape[0]
    gk, cnt, posf = pl.pallas_call(
        _router_body,
        grid=(1,),
        in_specs=[
            pl.BlockSpec((n, d), lambda i: (0, 0)),
            pl.BlockSpec((d, e), lambda i: (0, 0)),
            pl.BlockSpec((1, e), lambda i: (0, 0)),
        ],
        out_specs=[
            pl.BlockSpec((K_TOP, n), lambda i: (0, 0)),
            pl.BlockSpec((1, e), lambda i: (0, 0)),
            pl.BlockSpec((K_TOP, n), lambda i: (0, 0)),
        ],
        out_shape=[
            jax.ShapeDtypeStruct((K_TOP, n), jnp.float32),
            jax.ShapeDtypeStruct((1, e), jnp.int32),
            jax.ShapeDtypeStruct((K_TOP, n), jnp.float32),
        ],
    )(xf, Wr.T, br.reshape(1, e))
    return gk, cnt, posf.astype(jnp.int32)


def _sc_dispatch(xf, idx):
    n, d = xf.shape
    np_ = K_TOP * n
    nchunk = idx.shape[1] // K_TOP
    ctok = n // (32 * nchunk)

    mesh = plsc.VectorSubcoreMesh(core_axis_name="c", subcore_axis_name="s")

    nslot = K_TOP * nchunk

    @functools.partial(
        pl.kernel, mesh=mesh,
        out_type=jax.ShapeDtypeStruct((np_, d), jnp.float32),
        scratch_types=(
            [pltpu.VMEM((ctok,), jnp.int32)] * nslot
            + [pltpu.VMEM((ctok, d), jnp.float32), pltpu.SemaphoreType.DMA]),
    )
    def k(xf_hbm, idx_hbm, xs_hbm, *refs):
        idx_vs, rows_v, sem = refs[:nslot], refs[nslot], refs[nslot + 1]
        nc = 2
        wid = lax.axis_index("s") * nc + lax.axis_index("c")
        for j in range(nslot):
            pltpu.sync_copy(idx_hbm.at[wid * nslot + j], idx_vs[j])
        for c in range(nchunk):
            base = wid * (nchunk * ctok) + c * ctok
            pltpu.sync_copy(xf_hbm.at[pl.ds(base, ctok)], rows_v)
            for kk in range(K_TOP):
                pltpu.async_copy(
                    rows_v, xs_hbm.at[idx_vs[K_TOP * c + kk]], sem).wait()

    return k(xf, idx.reshape(32 * nslot, ctok))


def _sc_gather_pairs(ysorted, idx):
    np_, d = ysorted.shape
    n = np_ // K_TOP
    nchunk = idx.shape[1] // K_TOP
    ctok = n // (32 * nchunk)

    mesh = plsc.VectorSubcoreMesh(core_axis_name="c", subcore_axis_name="s")

    nslot = K_TOP * nchunk

    @functools.partial(
        pl.kernel, mesh=mesh,
        out_type=jax.ShapeDtypeStruct((K_TOP * n, d), jnp.float32),
        scratch_types=(
            [pltpu.VMEM((ctok,), jnp.int32)] * nslot
            + [pltpu.VMEM((ctok, d), jnp.float32), pltpu.SemaphoreType.DMA]),
    )
    def k(ys_hbm, idx_hbm, yg_hbm, *refs):
        idx_vs, rows_v, sem = refs[:nslot], refs[nslot], refs[nslot + 1]
        nc = 2
        wid = lax.axis_index("s") * nc + lax.axis_index("c")
        for j in range(nslot):
            pltpu.sync_copy(idx_hbm.at[wid * nslot + j], idx_vs[j])
        for c in range(nchunk):
            base = wid * (nchunk * ctok) + c * ctok
            for kk in range(K_TOP):
                pltpu.async_copy(
                    ys_hbm.at[idx_vs[K_TOP * c + kk]], rows_v, sem).wait()
                pltpu.sync_copy(rows_v, yg_hbm.at[pl.ds(kk * n + base, ctok)])

    return k(ysorted, idx.reshape(32 * nslot, ctok)).reshape(K_TOP, n, d)


def _gmm_body(tile_ref, exp_ref, start_ref, end_ref,
              xs_ref, w1_ref, b1_ref, w2_ref, b2_ref, y_ref):
    g = pl.program_id(0)
    a = jnp.dot(xs_ref[...], w1_ref[0].T, preferred_element_type=jnp.float32)
    a = a + b1_ref[0]
    h = a * jax.nn.sigmoid(a)
    y = lax.dot_general(h, w2_ref[0], (((1,), (1,)), ((), ())),
                        preferred_element_type=jnp.float32)
    y = y + b2_ref[0]
    row = tile_ref[g] * TM + lax.broadcasted_iota(jnp.int32, (TM, 1), 0)
    mask = (row >= start_ref[g]) & (row < end_ref[g])
    y_ref[...] = jnp.where(mask, y, y_ref[...])


def _gmm(seg_tile, seg_exp, seg_start, seg_end, xs, W1, b1, W2, b2):
    np_, d = xs.shape
    e, m, _ = W1.shape
    gseg = seg_tile.shape[0]
    grid_spec = pltpu.PrefetchScalarGridSpec(
        num_scalar_prefetch=4,
        grid=(gseg,),
        in_specs=[
            pl.BlockSpec((TM, d), lambda g, st, se, ss, sn: (st[g], 0)),
            pl.BlockSpec((1, m, d), lambda g, st, se, ss, sn: (se[g], 0, 0)),
            pl.BlockSpec((1, 1, m), lambda g, st, se, ss, sn: (se[g], 0, 0)),
            pl.BlockSpec((1, d, m), lambda g, st, se, ss, sn: (se[g], 0, 0)),
            pl.BlockSpec((1, 1, d), lambda g, st, se, ss, sn: (se[g], 0, 0)),
        ],
        out_specs=pl.BlockSpec((TM, d), lambda g, st, se, ss, sn: (st[g], 0)),
    )
    return pl.pallas_call(
        _gmm_body,
        grid_spec=grid_spec,
        out_shape=jax.ShapeDtypeStruct((np_, d), jnp.float32),
    )(seg_tile, seg_exp, seg_start, seg_end,
      xs, W1, b1[:, None, :], W2, b2[:, None, :])


def _shared_body(x_ref, w1_ref, b1_ref, w2_ref, b2_ref, out_ref, *, tm):
    p, mt, t = pl.program_id(0), pl.program_id(1), pl.program_id(2)
    rows = pl.ds(t * tm, tm)
    a = jnp.dot(x_ref[rows, :], w1_ref[0].T,
                preferred_element_type=jnp.float32)
    a = a + b1_ref[0]
    h = a * jax.nn.sigmoid(a)
    y = lax.dot_general(h, w2_ref[0], (((1,), (1,)), ((), ())),
                        preferred_element_type=jnp.float32)

    @pl.when((p == 0) & (mt == 0))
    def _():
        out_ref[rows, :] = y + b2_ref[0]

    @pl.when((p > 0) & (mt == 0))
    def _():
        out_ref[rows, :] += y + b2_ref[0]

    @pl.when(mt > 0)
    def _():
        out_ref[rows, :] += y


def _shared(xf, W1s, b1s, W2s, b2s):
    n, d = xf.shape
    p, m, _ = W1s.shape
    tn = min(1024, m)
    tm = min(1024, n)
    mt = m // tn
    return pl.pallas_call(
        functools.partial(_shared_body, tm=tm),
        grid=(p, mt, n // tm),
        in_specs=[
            pl.BlockSpec((n, d), lambda p, mt, t: (0, 0)),
            pl.BlockSpec((1, tn, d), lambda p, mt, t: (p, mt, 0)),
            pl.BlockSpec((1, 1, tn), lambda p, mt, t: (p, 0, mt)),
            pl.BlockSpec((1, d, tn), lambda p, mt, t: (p, 0, mt)),
            pl.BlockSpec((1, 1, d), lambda p, mt, t: (p, 0, 0)),
        ],
        out_specs=pl.BlockSpec((n, d), lambda p, mt, t: (0, 0)),
        out_shape=jax.ShapeDtypeStruct((n, d), jnp.float32),
    )(xf, W1s, b1s[:, None, :], W2s, b2s[:, None, :])


def _combine_body(yg_ref, sh_ref, gk_ref, out_ref):
    g0 = gk_ref[0, 0, :]
    g1 = gk_ref[1, 0, :]
    out_ref[...] = (g0[:, None] * yg_ref[0] + g1[:, None] * yg_ref[1]
                    + sh_ref[...])


def _combine(yg, sh, gk):
    _, n, d = yg.shape
    tm = 512
    return pl.pallas_call(
        _combine_body,
        grid=(n // tm,),
        in_specs=[
            pl.BlockSpec((K_TOP, tm, d), lambda t: (0, t, 0)),
            pl.BlockSpec((tm, d), lambda t: (t, 0)),
            pl.BlockSpec((K_TOP, 1, tm), lambda t: (0, 0, t)),
        ],
        out_specs=pl.BlockSpec((tm, d), lambda t: (t, 0)),
        out_shape=jax.ShapeDtypeStruct((n, d), jnp.float32),
    )(yg, sh, gk)


def _segments(cnt, n_pairs):
    e = cnt.shape[0]
    t = n_pairs // TM
    gseg = t + e - 1
    off = jnp.concatenate([jnp.zeros((1,), jnp.int32),
                           jnp.cumsum(cnt, dtype=jnp.int32)])
    first = off[:-1] // TM
    last = jnp.where(cnt > 0, (off[1:] - 1) // TM, first - 1)
    n_e = jnp.maximum(last - first + 1, 0)
    base = jnp.concatenate([jnp.zeros((1,), jnp.int32),
                            jnp.cumsum(n_e, dtype=jnp.int32)])
    gs = jnp.arange(gseg, dtype=jnp.int32)
    eg = jnp.sum((base[1:][None, :] <= gs[:, None]).astype(jnp.int32), axis=1)
    eg = jnp.minimum(eg, e - 1)
    valid = gs < base[e]
    tile = jnp.where(valid, first[eg] + (gs - base[eg]), t - 1)
    start = jnp.where(valid, jnp.maximum(off[eg], tile * TM), 0)
    end = jnp.where(valid, jnp.minimum(off[eg + 1], (tile + 1) * TM), 0)
    return tile, eg, start, end


def kernel(x, Wr, br, W1, b1, W2, b2, Ws1, bs1, Ws2, bs2):
    b, s, d = x.shape
    e, m, _ = W1.shape
    ns, i, _ = Ws1.shape
    n = b * s
    xf = x.reshape(n, d)

    gk, cnt, pos = _router(xf, Wr, br)
    cnt = cnt[0]

    nchunk = 2
    ctok = n // (32 * nchunk)
    idx = jnp.stack([pos[0].reshape(32, nchunk, ctok),
                     pos[1].reshape(32, nchunk, ctok)], axis=2)
    idx = idx.reshape(32, K_TOP * nchunk, ctok)

    xs = _sc_dispatch(xf, idx)

    seg_tile, seg_exp, seg_start, seg_end = _segments(cnt, K_TOP * n)
    ys = _gmm(seg_tile, seg_exp, seg_start, seg_end, xs, W1, b1, W2, b2)

    yg = _sc_gather_pairs(ys, idx)

    nc2 = i // m
    W1s = Ws1.reshape(ns * nc2, m, d)
    b1s = bs1.reshape(ns * nc2, m)
    W2s = Ws2.reshape(ns, d, nc2, m).transpose(0, 2, 1, 3).reshape(
        ns * nc2, d, m)
    bs2_chunks = jnp.concatenate(
        [bs2[:, None, :], jnp.zeros((ns, nc2 - 1, d), bs2.dtype)], axis=1)
    b2s = bs2_chunks.reshape(ns * nc2, d)
    sh = _shared(xf, W1s, b1s, W2s, b2s)

    out = _combine(yg, sh, gk[:, None, :])
    return out.reshape(b, s, d)
